# baseline (speedup 1.0000x reference)
import functools

import jax
import jax.numpy as jnp
from jax import lax
from jax.experimental import pallas as pl
from jax.experimental.pallas import tpu as pltpu
from jax.experimental.pallas import tpu_sc as plsc

N = 10000
E = 320000
D = 128
NCLS = 32

NC = 2
NS = 16
NW = NC * NS
EPW = E // NW
CHUNK = 80
NCHUNK = EPW // CHUNK
BLKC = 25
NBLK = NCHUNK // BLKC
NPAD = 10240
ROWS_PT = NPAD // NS
DEGW = 128
WROWS = CHUNK * 16 // 128


def _sc_segment(x, src, dst, w, z128):
    mesh = plsc.VectorSubcoreMesh(core_axis_name="c", subcore_axis_name="s")

    @functools.partial(
        pl.kernel,
        out_type=[
            jax.ShapeDtypeStruct((NC, NPAD, D), jnp.float32),
            jax.ShapeDtypeStruct((NC, NPAD, DEGW), jnp.float32),
        ],
        mesh=mesh,
        scratch_types=[
            pltpu.VMEM((BLKC, CHUNK), jnp.int32),
            pltpu.VMEM((BLKC, CHUNK), jnp.int32),
            pltpu.VMEM((16, 128), jnp.float32),
            pltpu.VMEM((CHUNK, D), jnp.float32),
            pltpu.VMEM((CHUNK, D), jnp.float32),
            pltpu.VMEM_SHARED((NPAD, D), jnp.float32),
            pltpu.SemaphoreType.DMA,
            pltpu.SemaphoreType.DMA,
            pltpu.SemaphoreType.DMA,
            pltpu.SemaphoreType.DMA,
        ],
    )
    def k(x_hbm, src_hbm, dst_hbm, w_hbm, z128_hbm, agg_out, deg_out,
          src_v, dst_v, w_blk, rows_a, rows_b, agg_sh,
          sem_ga, sem_gb, sem_sa, sem_sb):
        c = lax.axis_index("c")
        s = lax.axis_index("s")
        wid = c * NS + s
        base = s * ROWS_PT

        rows = (rows_a, rows_b)
        gsem = (sem_ga, sem_gb)
        ssem = (sem_sa, sem_sb)

        def start_fetch(j, p):
            return pltpu.async_copy(
                x_hbm.at[src_v.at[lax.rem(j, BLKC)]], rows[p], gsem[p])

        def scale_rows(j, p):
            jb = lax.rem(j, BLKC)

            def grp_body(g, carry3):
                e0 = jb * CHUNK + g * 16
                wgrp = w_blk[e0 // 128, pl.ds(lax.rem(e0, 128), 16)]
                for k in range(16):
                    wv = lax.gather(
                        wgrp, jnp.full((16, 1), k, jnp.int32),
                        lax.GatherDimensionNumbers(
                            offset_dims=(), collapsed_slice_dims=(0,),
                            start_index_map=(0,)),
                        (1,), mode=lax.GatherScatterMode.PROMISE_IN_BOUNDS)
                    r = g * 16 + k
                    for t in range(D // 16):
                        sl = pl.ds(t * 16, 16)
                        rows[p][r, sl] = rows[p][r, sl] * wv
                return carry3

            lax.fori_loop(0, CHUNK // 16, grp_body, 0)

        def process(j, p, sprev, sync_sc, last=False):
            if sprev is not None:
                sprev.wait()

            @pl.when((lax.rem(j, BLKC) == 0) & (j > 0))
            def _():
                pltpu.sync_copy(dst_hbm.at[wid, j // BLKC], dst_v)
                pltpu.sync_copy(w_hbm.at[wid, j // BLKC], w_blk)

            descs = None
            if not last:
                jn = j + 1

                @pl.when(lax.rem(jn, BLKC) == 0)
                def _():
                    pltpu.sync_copy(src_hbm.at[wid, jn // BLKC], src_v)

                descs = start_fetch(jn, 1 - p)

            scale_rows(j, p)
            tgt = agg_sh.at[dst_v.at[lax.rem(j, BLKC)]]
            if sync_sc:
                pltpu.sync_copy(rows[p], tgt, add=True)
                sc = None
            else:
                sc = pltpu.async_copy(rows[p], tgt, ssem[p], add=True)
            if descs is not None:
                descs.wait()
            return sc

        pltpu.sync_copy(z128_hbm.at[pl.ds(base, ROWS_PT)],
                        agg_sh.at[pl.ds(base, ROWS_PT)])
        pltpu.sync_copy(src_hbm.at[wid, 0], src_v)
        pltpu.sync_copy(dst_hbm.at[wid, 0], dst_v)
        pltpu.sync_copy(w_hbm.at[wid, 0], w_blk)
        plsc.subcore_barrier()

        start_fetch(0, 0).wait()

        def quad_body(jq, carry):
            j0 = jq * 4
            s0 = process(j0, 0, None, sync_sc=False)
            s1 = process(j0 + 1, 1, s0, sync_sc=False)
            s2 = process(j0 + 2, 0, s1, sync_sc=False)
            process(j0 + 3, 1, s2, sync_sc=True)
            return carry

        lax.fori_loop(0, (NCHUNK - 1) // 4, quad_body, 0)
        process(NCHUNK - 1, 0, None, sync_sc=True, last=True)

        plsc.subcore_barrier()
        pltpu.sync_copy(agg_sh.at[pl.ds(base, ROWS_PT)],
                        agg_out.at[c, pl.ds(base, ROWS_PT)])
        plsc.subcore_barrier()

        pltpu.sync_copy(z128_hbm.at[pl.ds(base, ROWS_PT)],
                        agg_sh.at[pl.ds(base, ROWS_PT)])

        onev = jnp.ones((16,), jnp.float32)

        def ones_body(r, carry):
            for t in range(D // 16):
                rows_a[r, pl.ds(t * 16, 16)] = onev
            return carry

        lax.fori_loop(0, CHUNK, ones_body, 0)
        plsc.subcore_barrier()

        def deg_blk(b, carry):
            pltpu.sync_copy(dst_hbm.at[wid, b], dst_v)

            def deg_grp(g, carry2):
                descs = [
                    pltpu.async_copy(rows_a, agg_sh.at[dst_v.at[g * 5 + u]],
                                     sem_sa, add=True)
                    for u in range(5)
                ]
                for d in descs:
                    d.wait()
                return carry2

            lax.fori_loop(0, BLKC // 5, deg_grp, 0)
            return carry

        lax.fori_loop(0, NBLK, deg_blk, 0)

        plsc.subcore_barrier()
        pltpu.sync_copy(agg_sh.at[pl.ds(base, ROWS_PT)],
                        deg_out.at[c, pl.ds(base, ROWS_PT)])

    return k(x, src, dst, w, z128)


def _tc_body(agg_ref, deg_ref, w1_ref, b1_ref, w2_ref, b2_ref, out_ref):
    agg = agg_ref[0] + agg_ref[1]
    deg = deg_ref[0, :, 0:1] + deg_ref[1, :, 0:1]
    h = jnp.tanh(agg / jnp.maximum(deg, 1.0))
    m = jnp.sum(h, axis=0, keepdims=True) * (1.0 / N)
    p = jnp.dot(m, w1_ref[...], preferred_element_type=jnp.float32) + b1_ref[...]
    out_ref[...] = (
        jnp.dot(p, w2_ref[...], preferred_element_type=jnp.float32) + b2_ref[...]
    )


def _tc_finish(agg_p, deg_p, W1, b1, W2, b2):
    return pl.pallas_call(
        _tc_body,
        out_shape=jax.ShapeDtypeStruct((1, NCLS), jnp.float32),
    )(agg_p, deg_p, W1, b1, W2, b2)


def kernel(x, edge_index, edge_w, W1, b1, W2, b2):
    src = edge_index[0].reshape(NW, NBLK, BLKC, CHUNK)
    dst = edge_index[1].reshape(NW, NBLK, BLKC, CHUNK)
    wb = jnp.pad(edge_w.reshape(NW, NBLK, BLKC * CHUNK), ((0, 0), (0, 0), (0, 48)))
    wb = wb.reshape(NW, NBLK, 16, 128)
    z128 = jnp.zeros((NPAD, D), jnp.float32)
    agg_p, deg_p = _sc_segment(x, src, dst, wb, z128)
    return _tc_finish(agg_p, deg_p, W1, b1.reshape(1, D), W2, b2.reshape(1, NCLS))
